# Initial kernel scaffold; baseline (speedup 1.0000x reference)
import jax
import jax.numpy as jnp
from jax.experimental import pallas as pl


def kernel(x, question_mask):
    raise NotImplementedError("write your pallas kernel here")


# speedup vs baseline: 5.4728x; 5.4728x over previous
import functools
import numpy as np
import jax
import jax.numpy as jnp
from jax import lax
from jax.experimental import pallas as pl
from jax.experimental.pallas import tpu as pltpu
from jax.experimental.pallas import tpu_sc as plsc

_FR = [
    ((0.0, 0.33), (0.0, 0.33), (0.0, 0.33), 0.0),
    ((0.0, 0.33), (0.0, 0.33), (0.67, 1.0), 0.333),
    ((0.0, 0.33), (0.33, 0.67), (0.33, 0.67), 0.333),
    ((0.0, 0.33), (0.67, 1.0), (0.0, 0.33), 0.333),
    ((0.0, 0.33), (0.67, 1.0), (0.67, 1.0), 0.667),
    ((0.33, 0.67), (0.0, 0.33), (0.0, 0.33), 0.333),
    ((0.33, 0.67), (0.0, 0.33), (0.67, 1.0), 0.667),
    ((0.33, 0.67), (0.33, 0.67), (0.33, 0.67), 0.5),
    ((0.33, 0.67), (0.67, 1.0), (0.0, 0.33), 0.667),
    ((0.33, 0.67), (0.67, 1.0), (0.67, 1.0), 1.0),
    ((0.67, 1.0), (0.0, 0.33), (0.0, 0.33), 0.667),
    ((0.67, 1.0), (0.0, 0.33), (0.67, 1.0), 1.0),
    ((0.67, 1.0), (0.33, 0.67), (0.33, 0.67), 1.0),
    ((0.67, 1.0), (0.67, 1.0), (0.0, 0.33), 1.167),
    ((0.67, 1.0), (0.67, 1.0), (0.67, 1.0), 1.5),
]
_LO = np.array([[r[0][0], r[1][0], r[2][0]] for r in _FR], dtype=np.float32)
_HI = np.array([[r[0][1], r[1][1], r[2][1]] for r in _FR], dtype=np.float32)
_CONS = np.array([r[3] for r in _FR], dtype=np.float32)
_CEN = (_LO + _HI) / 2.0


def _ent_body(x_ref, out_ent_ref, x3_ref, mtxt_ref, ent_s, sum_s, img_s,
              *, B, S, D, BS, NBLK):
    b = pl.program_id(0)
    j = pl.program_id(1)

    xb = x_ref[0]
    m = jnp.max(xb, axis=1, keepdims=True)
    t = xb - m
    e = jnp.exp(t)
    Z = jnp.sum(e, axis=1, keepdims=True)
    sxe = jnp.sum(e * t, axis=1, keepdims=True)
    ent = jnp.log(Z) - sxe / Z

    ent_s[pl.ds(b, 1), pl.ds(j * BS, BS)] = ent.reshape(1, BS)

    part = jnp.sum(xb, axis=0, keepdims=True)

    @pl.when(j == 0)
    def _():
        sum_s[pl.ds(b, 1), :] = part
        img_s[pl.ds(b, 1), :] = xb[0:1, :]

    @pl.when(j != 0)
    def _():
        sum_s[pl.ds(b, 1), :] = sum_s[pl.ds(b, 1), :] + part

    @pl.when((b == B - 1) & (j == NBLK - 1))
    def _():
        ent_all = ent_s[:, :]
        img = img_s[:, :]
        tot = sum_s[:, :]
        out_ent_ref[:, :] = ent_all

        cols = jax.lax.broadcasted_iota(jnp.int32, (B, S), 1)
        ent_txt = jnp.where(cols >= 1, ent_all, -jnp.inf)
        mtxt_ref[:, :] = jnp.reshape(jnp.max(ent_txt), (1, 1))

        bmean = (tot - img) / float(S - 1)
        dot = jnp.sum(img * bmean, axis=1, keepdims=True)
        na = jnp.sqrt(jnp.sum(img * img, axis=1, keepdims=True))
        nb = jnp.sqrt(jnp.sum(bmean * bmean, axis=1, keepdims=True))
        x3_ref[:, :] = dot / jnp.clip(na * nb, 1e-8, None)


def _nsqrt(d):
    bits = plsc.bitcast(d, jnp.int32)
    s = plsc.bitcast((bits >> 1) + 0x1FBD1DF5, jnp.float32)
    for _ in range(3):
        s = 0.5 * (s + d / jnp.maximum(s, 1e-30))
    return jnp.where(d > 0.0, s, 0.0)


def _route_body(ent_hbm, params_hbm, f_hbm, g_hbm, ent_v, par_v, f_v, g_v,
                *, TPW, S):
    c = lax.axis_index("c")
    s = lax.axis_index("s")
    wid = c * 16 + s
    base = wid * TPW
    pltpu.sync_copy(ent_hbm.at[pl.ds(base, TPW)], ent_v)
    pltpu.sync_copy(params_hbm.at[wid], par_v)
    x1 = par_v[pl.ds(0, 16)]
    x3 = par_v[pl.ds(16, 16)]
    rtx = par_v[pl.ds(32, 16)]
    lane = lax.iota(jnp.int32, 16)
    for j in range(TPW // 16):
        e = ent_v[pl.ds(j * 16, 16)]
        x2 = e * rtx
        any_m = jnp.zeros((16,), jnp.bool_)
        mval = jnp.zeros((16,), jnp.float32)
        d1 = jnp.full((16,), jnp.inf, jnp.float32)
        d2 = jnp.full((16,), jnp.inf, jnp.float32)
        c1 = jnp.zeros((16,), jnp.float32)
        c2 = jnp.zeros((16,), jnp.float32)
        for r in range(15):
            lo0, lo1, lo2 = (float(v) for v in _LO[r])
            hi0, hi1, hi2 = (float(v) for v in _HI[r])
            ce0, ce1, ce2 = (float(v) for v in _CEN[r])
            cons = float(_CONS[r])
            m_r = ((x1 >= lo0) & (x1 < hi0)
                   & (x2 >= lo1) & (x2 < hi1)
                   & (x3 >= lo2) & (x3 < hi2))
            any_m = any_m | m_r
            mval = mval + jnp.where(m_r, cons, 0.0)
            dx = x1 - ce0
            dy = x2 - ce1
            dz = x3 - ce2
            dq = dx * dx + dy * dy + dz * dz
            lt1 = dq < d1
            lt2 = (dq < d2) & (~lt1)
            d2n = jnp.where(lt1, d1, jnp.where(lt2, dq, d2))
            c2n = jnp.where(lt1, c1, jnp.where(lt2, cons, c2))
            d1 = jnp.where(lt1, dq, d1)
            c1 = jnp.where(lt1, cons, c1)
            d2 = d2n
            c2 = c2n
        s1 = _nsqrt(d1)
        s2 = _nsqrt(d2)
        dsum = s1 + s2
        lam = jnp.where(dsum != 0.0,
                        s1 / jnp.where(dsum == 0.0, 1.0, dsum), 0.5)
        interp = (1.0 - lam) * c1 + lam * c2
        f = jnp.where(any_m, mval, interp)
        tid = base + j * 16 + lane
        tmask = (tid % S) == 0
        f = jnp.where(tmask, 0.0, f)
        g = jnp.where(tmask, 0.0, 1.0 - f)
        f_v[pl.ds(j * 16, 16)] = f
        g_v[pl.ds(j * 16, 16)] = g
    pltpu.sync_copy(f_v, f_hbm.at[pl.ds(base, TPW)])
    pltpu.sync_copy(g_v, g_hbm.at[pl.ds(base, TPW)])


def kernel(x, question_mask):
    B, S, D = x.shape
    BS = 512
    NBLK = S // BS
    image_tokens = x[:, 0:1, :]
    ipb = jax.nn.softmax(image_tokens, axis=-1)
    ient = -(ipb * jnp.log(ipb + 1e-08)).sum(axis=-1)
    x1 = ient / jnp.clip(ient.max(), 1e-06, None)

    ent_body = functools.partial(_ent_body, B=B, S=S, D=D, BS=BS, NBLK=NBLK)
    ent, x3, mtxt = pl.pallas_call(
        ent_body,
        grid=(B, NBLK),
        in_specs=[pl.BlockSpec((1, BS, D), lambda b, j: (b, j, 0))],
        out_specs=[
            pl.BlockSpec((B, S), lambda b, j: (0, 0)),
            pl.BlockSpec((B, 1), lambda b, j: (0, 0)),
            pl.BlockSpec((1, 1), lambda b, j: (0, 0)),
        ],
        out_shape=[
            jax.ShapeDtypeStruct((B, S), jnp.float32),
            jax.ShapeDtypeStruct((B, 1), jnp.float32),
            jax.ShapeDtypeStruct((1, 1), jnp.float32),
        ],
        scratch_shapes=[
            pltpu.VMEM((B, S), jnp.float32),
            pltpu.VMEM((B, D), jnp.float32),
            pltpu.VMEM((B, D), jnp.float32),
        ],
    )(x)

    NW = 32
    TPW = (B * S) // NW
    rtx = 1.0 / jnp.maximum(mtxt[0, 0], 1e-6)
    wpb = NW // B
    x1w = jnp.repeat(x1.reshape(B), wpb)
    x3w = jnp.repeat(x3.reshape(B), wpb)
    params = jnp.concatenate([
        jnp.broadcast_to(x1w[:, None], (NW, 16)),
        jnp.broadcast_to(x3w[:, None], (NW, 16)),
        jnp.full((NW, 16), 1.0, jnp.float32) * rtx,
    ], axis=1)

    mesh = plsc.VectorSubcoreMesh(core_axis_name="c", subcore_axis_name="s")
    route = functools.partial(
        pl.kernel,
        out_type=(
            jax.ShapeDtypeStruct((B * S,), jnp.float32),
            jax.ShapeDtypeStruct((B * S,), jnp.float32),
        ),
        mesh=mesh,
        compiler_params=pltpu.CompilerParams(needs_layout_passes=False),
        scratch_types=[
            pltpu.VMEM((TPW,), jnp.float32),
            pltpu.VMEM((48,), jnp.float32),
            pltpu.VMEM((TPW,), jnp.float32),
            pltpu.VMEM((TPW,), jnp.float32),
        ],
    )(functools.partial(_route_body, TPW=TPW, S=S))
    f, g = route(ent.reshape(B * S), params)
    return jnp.stack([f.reshape(B, S), g.reshape(B, S)], axis=-1).astype(x.dtype)


# speedup vs baseline: 12.1028x; 2.2114x over previous
import numpy as np
import jax
import jax.numpy as jnp
from jax.experimental import pallas as pl
from jax.experimental.pallas import tpu as pltpu

_FR = [
    ((0.0, 0.33), (0.0, 0.33), (0.0, 0.33), 0.0),
    ((0.0, 0.33), (0.0, 0.33), (0.67, 1.0), 0.333),
    ((0.0, 0.33), (0.33, 0.67), (0.33, 0.67), 0.333),
    ((0.0, 0.33), (0.67, 1.0), (0.0, 0.33), 0.333),
    ((0.0, 0.33), (0.67, 1.0), (0.67, 1.0), 0.667),
    ((0.33, 0.67), (0.0, 0.33), (0.0, 0.33), 0.333),
    ((0.33, 0.67), (0.0, 0.33), (0.67, 1.0), 0.667),
    ((0.33, 0.67), (0.33, 0.67), (0.33, 0.67), 0.5),
    ((0.33, 0.67), (0.67, 1.0), (0.0, 0.33), 0.667),
    ((0.33, 0.67), (0.67, 1.0), (0.67, 1.0), 1.0),
    ((0.67, 1.0), (0.0, 0.33), (0.0, 0.33), 0.667),
    ((0.67, 1.0), (0.0, 0.33), (0.67, 1.0), 1.0),
    ((0.67, 1.0), (0.33, 0.67), (0.33, 0.67), 1.0),
    ((0.67, 1.0), (0.67, 1.0), (0.0, 0.33), 1.167),
    ((0.67, 1.0), (0.67, 1.0), (0.67, 1.0), 1.5),
]
_LO = np.array([[r[0][0], r[1][0], r[2][0]] for r in _FR], dtype=np.float32)
_HI = np.array([[r[0][1], r[1][1], r[2][1]] for r in _FR], dtype=np.float32)
_CONS = np.array([r[3] for r in _FR], dtype=np.float32)
_CEN = (_LO + _HI) / 2.0


def _body(x_ref, out0_ref, out1_ref, ent_s, sum_s, img_s, *, B, S, D, BS, NBLK):
    b = pl.program_id(0)
    j = pl.program_id(1)

    xb = x_ref[0]
    m = jnp.max(xb, axis=1, keepdims=True)
    t = xb - m
    e = jnp.exp(t)
    Z = jnp.sum(e, axis=1, keepdims=True)
    sxe = jnp.sum(e * t, axis=1, keepdims=True)
    ent = jnp.log(Z) - sxe / Z

    ent_s[pl.ds(b, 1), pl.ds(j * BS, BS)] = ent.reshape(1, BS)

    part = jnp.sum(xb, axis=0, keepdims=True)

    @pl.when(j == 0)
    def _():
        sum_s[pl.ds(b, 1), :] = part
        img_s[pl.ds(b, 1), :] = xb[0:1, :]

    @pl.when(j != 0)
    def _():
        sum_s[pl.ds(b, 1), :] = sum_s[pl.ds(b, 1), :] + part

    @pl.when((b == B - 1) & (j == NBLK - 1))
    def _():
        ent_all = ent_s[:, :]
        img = img_s[:, :]
        tot = sum_s[:, :]

        im = jnp.max(img, axis=1, keepdims=True)
        ie = jnp.exp(img - im)
        iZ = jnp.sum(ie, axis=1, keepdims=True)
        irZ = 1.0 / iZ
        ip = ie * irZ
        ent_img = -jnp.sum(ip * jnp.log(ip + 1e-08), axis=1, keepdims=True)
        mimg = jnp.max(ent_img)
        x1 = ent_img * (1.0 / jnp.maximum(mimg, 1e-6))

        cols = jax.lax.broadcasted_iota(jnp.int32, (B, S), 1)
        is_text = cols >= 1
        ent_txt = jnp.where(is_text, ent_all, -jnp.inf)
        mtxt = jnp.max(ent_txt)
        x2 = ent_all * (1.0 / jnp.maximum(mtxt, 1e-6))

        bmean = (tot - img) / float(S - 1)
        dot = jnp.sum(img * bmean, axis=1, keepdims=True)
        na = jnp.sqrt(jnp.sum(img * img, axis=1, keepdims=True))
        nb = jnp.sqrt(jnp.sum(bmean * bmean, axis=1, keepdims=True))
        x3 = dot / jnp.clip(na * nb, 1e-8, None)

        any_m = jnp.zeros((B, S), dtype=jnp.bool_)
        mval = jnp.zeros((B, S), dtype=jnp.float32)
        d1 = jnp.full((B, S), jnp.inf, dtype=jnp.float32)
        d2 = jnp.full((B, S), jnp.inf, dtype=jnp.float32)
        c1 = jnp.zeros((B, S), dtype=jnp.float32)
        c2 = jnp.zeros((B, S), dtype=jnp.float32)
        for r in range(15):
            lo0, lo1, lo2 = (float(v) for v in _LO[r])
            hi0, hi1, hi2 = (float(v) for v in _HI[r])
            ce0, ce1, ce2 = (float(v) for v in _CEN[r])
            cons = float(_CONS[r])
            m_r = ((x1 >= lo0) & (x1 < hi0)
                   & (x2 >= lo1) & (x2 < hi1)
                   & (x3 >= lo2) & (x3 < hi2))
            any_m = any_m | m_r
            mval = mval + jnp.where(m_r, cons, 0.0)
            dx = x1 - ce0
            dy = x2 - ce1
            dz = x3 - ce2
            d = jnp.sqrt(dx * dx + dy * dy + dz * dz)
            lt1 = d < d1
            lt2 = (d < d2) & (~lt1)
            d2n = jnp.where(lt1, d1, jnp.where(lt2, d, d2))
            c2n = jnp.where(lt1, c1, jnp.where(lt2, cons, c2))
            d1 = jnp.where(lt1, d, d1)
            c1 = jnp.where(lt1, cons, c1)
            d2 = d2n
            c2 = c2n

        dsum = d1 + d2
        lam = jnp.where(dsum != 0.0,
                        d1 / jnp.where(dsum == 0.0, 1.0, dsum), 0.5)
        interp = (1.0 - lam) * c1 + lam * c2
        f = jnp.where(any_m, mval, interp)
        f = jnp.where(is_text, f, 0.0)
        g = jnp.where(is_text, 1.0 - f, 0.0)
        out0_ref[:, :] = f
        out1_ref[:, :] = g


def kernel(x, question_mask):
    B, S, D = x.shape
    BS = 512
    NBLK = S // BS
    import functools
    body = functools.partial(_body, B=B, S=S, D=D, BS=BS, NBLK=NBLK)
    out0, out1 = pl.pallas_call(
        body,
        grid=(B, NBLK),
        in_specs=[pl.BlockSpec((1, BS, D), lambda b, j: (b, j, 0))],
        out_specs=[
            pl.BlockSpec((B, S), lambda b, j: (0, 0)),
            pl.BlockSpec((B, S), lambda b, j: (0, 0)),
        ],
        out_shape=[
            jax.ShapeDtypeStruct((B, S), jnp.float32),
            jax.ShapeDtypeStruct((B, S), jnp.float32),
        ],
        scratch_shapes=[
            pltpu.VMEM((B, S), jnp.float32),
            pltpu.VMEM((B, D), jnp.float32),
            pltpu.VMEM((B, D), jnp.float32),
        ],
    )(x)
    return jnp.stack([out0, out1], axis=-1).astype(x.dtype)
